# baseline (speedup 1.0000x reference)
import functools

import jax
import jax.numpy as jnp
from jax import lax
from jax.experimental import pallas as pl
from jax.experimental.pallas import tpu as pltpu
from jax.experimental.pallas import tpu_sc as plsc

_NC = 2
_NS = 16


@functools.partial(jax.jit, static_argnames=("n", "e", "d"))
def _sc_scatter_add(h, src, dst, *, n, e, d):
    nw = _NC * _NS
    epw = e // nw
    ch = 80
    nch = epw // ch
    assert nch * ch == epw
    NR = 4
    NI = 8
    rpt = (n // _NS) // 8 * 8
    nrem = n - _NS * rpt

    mesh = plsc.VectorSubcoreMesh(core_axis_name="c", subcore_axis_name="s")

    scratch = (
        [pltpu.VMEM((ch,), jnp.int32)] * NI +
        [pltpu.VMEM((ch,), jnp.int32)] * NI +
        [pltpu.VMEM((ch, d), jnp.float32)] * NR +
        [pltpu.VMEM_SHARED((n, d), jnp.float32)] +
        [pltpu.SemaphoreType.DMA] * NR +
        [pltpu.SemaphoreType.DMA] * NI +
        [pltpu.SemaphoreType.DMA] * NR
    )

    @functools.partial(
        pl.kernel,
        out_type=jax.ShapeDtypeStruct((_NC * n, d), jnp.float32),
        mesh=mesh,
        scratch_types=scratch,
    )
    def k(h_hbm, src_hbm, dst_hbm, out_hbm, *refs):
        srcb = refs[0:NI]
        dstb = refs[NI:2 * NI]
        rowsb = refs[2 * NI:2 * NI + NR]
        agg_sh = refs[2 * NI + NR]
        gsem = refs[2 * NI + NR + 1:2 * NI + 2 * NR + 1]
        isem = refs[2 * NI + 2 * NR + 1:3 * NI + 2 * NR + 1]
        ssem = refs[3 * NI + 2 * NR + 1:3 * NI + 3 * NR + 1]
        cid = lax.axis_index("c")
        sid = lax.axis_index("s")
        wid = cid * _NS + sid
        base0 = wid * epw

        def idx_load(c, s):
            base = base0 + c * ch
            pltpu.async_copy(src_hbm.at[pl.ds(base, ch)], srcb[s], isem[s])
            pltpu.async_copy(dst_hbm.at[pl.ds(base, ch)], dstb[s], isem[s])

        def idx_wait(s):
            pltpu.make_async_copy(src_hbm.at[pl.ds(0, ch)], srcb[s],
                                  isem[s]).wait()
            pltpu.make_async_copy(dst_hbm.at[pl.ds(0, ch)], dstb[s],
                                  isem[s]).wait()

        def gather(c_r, c_s):
            pltpu.async_copy(h_hbm.at[srcb[c_s]], rowsb[c_r], gsem[c_r])

        def gather_wait(c_r, c_s):
            pltpu.make_async_copy(h_hbm.at[srcb[c_s]], rowsb[c_r],
                                  gsem[c_r]).wait()

        def drain(c_r, c_s):
            pltpu.make_async_copy(rowsb[c_r], agg_sh.at[dstb[c_s]],
                                  ssem[c_r]).wait()

        def body(c, cm, *, do_drain, do_idx, do_gather):
            r, s = cm % NR, cm % NI
            if do_drain:
                drain((cm + 2) % NR, (cm + 6) % NI)
            if do_idx is None:
                @pl.when(c + 4 < nch)
                def _():
                    idx_load(c + 4, (cm + 4) % NI)
            elif do_idx:
                idx_load(c + 4, (cm + 4) % NI)
            if do_gather:
                idx_wait((cm + 2) % NI)
                gather((cm + 2) % NR, (cm + 2) % NI)
            gather_wait(r, s)
            pltpu.async_copy(rowsb[r], agg_sh.at[dstb[s]], ssem[r], add=True)

        for c0 in range(4):
            idx_load(c0, c0)
        for c0 in range(2):
            idx_wait(c0)
            gather(c0, c0)

        @pl.when(cid == 0)
        def _():
            pltpu.async_copy(h_hbm.at[pl.ds(sid * rpt, rpt)],
                             agg_sh.at[pl.ds(sid * rpt, rpt)], ssem[0])

            @pl.when(sid == _NS - 1)
            def _():
                pltpu.async_copy(h_hbm.at[pl.ds(_NS * rpt, nrem)],
                                 agg_sh.at[pl.ds(_NS * rpt, nrem)], ssem[0])
                pltpu.make_async_copy(
                    h_hbm.at[pl.ds(_NS * rpt, nrem)],
                    agg_sh.at[pl.ds(_NS * rpt, nrem)], ssem[0]).wait()

            pltpu.make_async_copy(h_hbm.at[pl.ds(sid * rpt, rpt)],
                                  agg_sh.at[pl.ds(sid * rpt, rpt)],
                                  ssem[0]).wait()

        @pl.when(cid == 1)
        def _():
            zeros16 = jnp.zeros((16,), jnp.float32)
            zbuf = rowsb[2]

            def zrow(r, _):
                for j in range(d // 16):
                    zbuf[r, pl.ds(j * 16, 16)] = zeros16
                return 0

            lax.fori_loop(0, ch, zrow, 0)
            nfull_z = rpt // ch
            for i in range(nfull_z):
                pltpu.sync_copy(zbuf,
                                agg_sh.at[pl.ds(sid * rpt + i * ch, ch)])
            zrem = rpt - nfull_z * ch
            if zrem:
                pltpu.sync_copy(
                    zbuf.at[pl.ds(0, zrem)],
                    agg_sh.at[pl.ds(sid * rpt + nfull_z * ch, zrem)])

            @pl.when(sid == _NS - 1)
            def _():
                pltpu.sync_copy(zbuf.at[pl.ds(0, nrem)],
                                agg_sh.at[pl.ds(_NS * rpt, nrem)])

        plsc.subcore_barrier()

        body(0, 0, do_drain=False, do_idx=True, do_gather=True)
        body(1, 1, do_drain=False, do_idx=True, do_gather=True)

        def octet(i, _):
            for b in range(8):
                c = 8 * i + 2 + b
                body(c, 2 + b, do_drain=True, do_idx=None, do_gather=True)
            return 0

        lax.fori_loop(0, (nch - 5) // 8, octet, 0)
        body(nch - 3, nch - 3, do_drain=True, do_idx=False, do_gather=True)
        body(nch - 2, nch - 2, do_drain=True, do_idx=False, do_gather=False)
        body(nch - 1, nch - 1, do_drain=True, do_idx=False, do_gather=False)
        drain((nch - 2) % NR, (nch - 2) % NI)
        drain((nch - 1) % NR, (nch - 1) % NI)

        plsc.subcore_barrier()

        pltpu.sync_copy(
            agg_sh.at[pl.ds(sid * rpt, rpt)],
            out_hbm.at[pl.ds(cid * n + sid * rpt, rpt)],
        )

        @pl.when(sid == _NS - 1)
        def _():
            pltpu.sync_copy(
                agg_sh.at[pl.ds(_NS * rpt, nrem)],
                out_hbm.at[pl.ds(cid * n + _NS * rpt, nrem)],
            )

    return k(h, src, dst)


def _tc_layer(agg, batch, p, *, n, d, hdim, g):
    eps = 1e-5

    def body(agg_ref, w1, b1, g1, be1, w2, b2, g2, be2, hout_ref):
        y = agg_ref[pl.ds(0, n), :] + agg_ref[pl.ds(n, n), :]
        z = jnp.dot(y, w1[...], preferred_element_type=jnp.float32) + b1[...]
        m = jnp.mean(z, axis=0)
        v = jnp.mean(z * z, axis=0) - m * m
        z = g1[...] * (z - m) * lax.rsqrt(v + eps) + be1[...]
        z = jnp.maximum(z, 0.0)
        z = jnp.dot(z, w2[...], preferred_element_type=jnp.float32) + b2[...]
        m2 = jnp.mean(z, axis=0)
        v2 = jnp.mean(z * z, axis=0) - m2 * m2
        z = g2[...] * (z - m2) * lax.rsqrt(v2 + eps) + be2[...]
        hout_ref[...] = jnp.maximum(z, 0.0)

    hn = pl.pallas_call(
        body,
        out_shape=jax.ShapeDtypeStruct((n, hdim), jnp.float32),
    )(agg, p["W1"], p["b1"], p["g1"], p["be1"],
      p["W2"], p["b2"], p["g2"], p["be2"])

    def pool_body(h_ref, b_ref, pool_ref):
        seg = lax.broadcasted_iota(jnp.int32, (g, n), 0)
        onehot = (seg == b_ref[...][None, :]).astype(jnp.float32)
        pool_ref[...] = jnp.dot(onehot, h_ref[...],
                                preferred_element_type=jnp.float32)

    pool = pl.pallas_call(
        pool_body,
        out_shape=jax.ShapeDtypeStruct((g, hdim), jnp.float32),
    )(hn, batch)
    return hn, pool


def kernel(x, edge_index, batch, params):
    n, d = x.shape
    e = edge_index.shape[1]
    g = 64
    src = edge_index[0]
    dst = edge_index[1]
    h = x
    pooled = []
    for p in params:
        hdim = p["W2"].shape[1]
        agg = _sc_scatter_add(h, src, dst, n=n, e=e, d=h.shape[1])
        h, pool = _tc_layer(agg, batch, p, n=n, d=h.shape[1], hdim=hdim, g=g)
        pooled.append(pool)
    return jnp.concatenate(pooled, axis=-1)


# speedup vs baseline: 1.0109x; 1.0109x over previous
import functools

import jax
import jax.numpy as jnp
from jax import lax
from jax.experimental import pallas as pl
from jax.experimental.pallas import tpu as pltpu
from jax.experimental.pallas import tpu_sc as plsc

_NC = 2
_NS = 16


@functools.partial(jax.jit, static_argnames=("n", "e", "d"))
def _sc_scatter_add(h, src, dst, *, n, e, d):
    nw = _NC * _NS
    epw = e // nw
    ch = 80
    nch = epw // ch
    assert nch * ch == epw
    NR = 4
    NI = 8
    rpt = (n // _NS) // 8 * 8
    nrem = n - _NS * rpt

    mesh = plsc.VectorSubcoreMesh(core_axis_name="c", subcore_axis_name="s")

    scratch = (
        [pltpu.VMEM((ch,), jnp.int32)] * NI +
        [pltpu.VMEM((ch,), jnp.int32)] * NI +
        [pltpu.VMEM((ch, d), jnp.float32)] * NR +
        [pltpu.VMEM_SHARED((n, d), jnp.float32)] +
        [pltpu.SemaphoreType.DMA] * NR +
        [pltpu.SemaphoreType.DMA] * NI +
        [pltpu.SemaphoreType.DMA] * NR
    )

    @functools.partial(
        pl.kernel,
        out_type=jax.ShapeDtypeStruct((_NC * n, d), jnp.float32),
        mesh=mesh,
        scratch_types=scratch,
    )
    def k(h_hbm, src_hbm, dst_hbm, out_hbm, *refs):
        srcb = refs[0:NI]
        dstb = refs[NI:2 * NI]
        rowsb = refs[2 * NI:2 * NI + NR]
        agg_sh = refs[2 * NI + NR]
        gsem = refs[2 * NI + NR + 1:2 * NI + 2 * NR + 1]
        isem = refs[2 * NI + 2 * NR + 1:3 * NI + 2 * NR + 1]
        ssem = refs[3 * NI + 2 * NR + 1:3 * NI + 3 * NR + 1]
        cid = lax.axis_index("c")
        sid = lax.axis_index("s")
        wid = cid * _NS + sid
        base0 = wid * epw

        def idx_load(c, s):
            base = base0 + c * ch
            pltpu.async_copy(src_hbm.at[pl.ds(base, ch)], srcb[s], isem[s])
            pltpu.async_copy(dst_hbm.at[pl.ds(base, ch)], dstb[s], isem[s])

        def idx_wait(s):
            pltpu.make_async_copy(src_hbm.at[pl.ds(0, ch)], srcb[s],
                                  isem[s]).wait()
            pltpu.make_async_copy(dst_hbm.at[pl.ds(0, ch)], dstb[s],
                                  isem[s]).wait()

        def gather(c_r, c_s):
            pltpu.async_copy(h_hbm.at[srcb[c_s]], rowsb[c_r], gsem[c_r])

        def gather_wait(c_r, c_s):
            pltpu.make_async_copy(h_hbm.at[srcb[c_s]], rowsb[c_r],
                                  gsem[c_r]).wait()

        def drain(c_r, c_s):
            pltpu.make_async_copy(rowsb[c_r], agg_sh.at[dstb[c_s]],
                                  ssem[c_r]).wait()

        def body(c, cm, *, do_drain, do_idx, do_gather):
            r, s = cm % NR, cm % NI
            if do_drain:
                drain((cm + 2) % NR, (cm + 6) % NI)
            if do_idx is None:
                @pl.when(c + 4 < nch)
                def _():
                    idx_load(c + 4, (cm + 4) % NI)
            elif do_idx:
                idx_load(c + 4, (cm + 4) % NI)
            if do_gather:
                idx_wait((cm + 2) % NI)
                gather((cm + 2) % NR, (cm + 2) % NI)
            gather_wait(r, s)
            pltpu.async_copy(rowsb[r], agg_sh.at[dstb[s]], ssem[r], add=True)

        for c0 in range(4):
            idx_load(c0, c0)
        for c0 in range(2):
            idx_wait(c0)
            gather(c0, c0)

        @pl.when(cid == 0)
        def _():
            pltpu.async_copy(h_hbm.at[pl.ds(sid * rpt, rpt)],
                             agg_sh.at[pl.ds(sid * rpt, rpt)], ssem[0])

            @pl.when(sid == _NS - 1)
            def _():
                pltpu.async_copy(h_hbm.at[pl.ds(_NS * rpt, nrem)],
                                 agg_sh.at[pl.ds(_NS * rpt, nrem)], ssem[0])
                pltpu.make_async_copy(
                    h_hbm.at[pl.ds(_NS * rpt, nrem)],
                    agg_sh.at[pl.ds(_NS * rpt, nrem)], ssem[0]).wait()

            pltpu.make_async_copy(h_hbm.at[pl.ds(sid * rpt, rpt)],
                                  agg_sh.at[pl.ds(sid * rpt, rpt)],
                                  ssem[0]).wait()

        @pl.when(cid == 1)
        def _():
            zeros16 = jnp.zeros((16,), jnp.float32)
            zbuf = rowsb[2]

            def zrow(r, _):
                for j in range(d // 16):
                    zbuf[r, pl.ds(j * 16, 16)] = zeros16
                return 0

            lax.fori_loop(0, ch, zrow, 0)
            nfull_z = rpt // ch
            for i in range(nfull_z):
                pltpu.sync_copy(zbuf,
                                agg_sh.at[pl.ds(sid * rpt + i * ch, ch)])
            zrem = rpt - nfull_z * ch
            if zrem:
                pltpu.sync_copy(
                    zbuf.at[pl.ds(0, zrem)],
                    agg_sh.at[pl.ds(sid * rpt + nfull_z * ch, zrem)])

            @pl.when(sid == _NS - 1)
            def _():
                pltpu.sync_copy(zbuf.at[pl.ds(0, nrem)],
                                agg_sh.at[pl.ds(_NS * rpt, nrem)])

        plsc.subcore_barrier()

        body(0, 0, do_drain=False, do_idx=True, do_gather=True)
        body(1, 1, do_drain=False, do_idx=True, do_gather=True)

        def octet(i, _):
            for b in range(8):
                c = 8 * i + 2 + b
                body(c, 2 + b, do_drain=True, do_idx=None, do_gather=True)
            return 0

        lax.fori_loop(0, (nch - 5) // 8, octet, 0)
        body(nch - 3, nch - 3, do_drain=True, do_idx=False, do_gather=True)
        body(nch - 2, nch - 2, do_drain=True, do_idx=False, do_gather=False)
        body(nch - 1, nch - 1, do_drain=True, do_idx=False, do_gather=False)
        drain((nch - 2) % NR, (nch - 2) % NI)
        drain((nch - 1) % NR, (nch - 1) % NI)

        plsc.subcore_barrier()

        pltpu.sync_copy(
            agg_sh.at[pl.ds(sid * rpt, rpt)],
            out_hbm.at[pl.ds(cid * n + sid * rpt, rpt)],
        )

        @pl.when(sid == _NS - 1)
        def _():
            pltpu.sync_copy(
                agg_sh.at[pl.ds(_NS * rpt, nrem)],
                out_hbm.at[pl.ds(cid * n + _NS * rpt, nrem)],
            )

    return k(h, src, dst)


def _tc_layer(agg, batch, p, *, n, d, hdim, g):
    eps = 1e-5

    def body(agg_ref, b_ref, w1, b1, g1, be1, w2, b2, g2, be2,
             hout_ref, pool_ref):
        y = agg_ref[pl.ds(0, n), :] + agg_ref[pl.ds(n, n), :]
        z = jnp.dot(y, w1[...], preferred_element_type=jnp.float32) + b1[...]
        m = jnp.mean(z, axis=0)
        v = jnp.mean(z * z, axis=0) - m * m
        z = g1[...] * (z - m) * lax.rsqrt(v + eps) + be1[...]
        z = jnp.maximum(z, 0.0)
        z = jnp.dot(z, w2[...], preferred_element_type=jnp.float32) + b2[...]
        m2 = jnp.mean(z, axis=0)
        v2 = jnp.mean(z * z, axis=0) - m2 * m2
        z = g2[...] * (z - m2) * lax.rsqrt(v2 + eps) + be2[...]
        hn = jnp.maximum(z, 0.0)
        hout_ref[...] = hn
        seg = lax.broadcasted_iota(jnp.int32, (g, n), 0)
        onehot = (seg == b_ref[...][None, :]).astype(jnp.float32)
        pool_ref[...] = jnp.dot(onehot, hn, preferred_element_type=jnp.float32)

    return pl.pallas_call(
        body,
        out_shape=(
            jax.ShapeDtypeStruct((n, hdim), jnp.float32),
            jax.ShapeDtypeStruct((g, hdim), jnp.float32),
        ),
    )(agg, batch, p["W1"], p["b1"], p["g1"], p["be1"],
      p["W2"], p["b2"], p["g2"], p["be2"])


def kernel(x, edge_index, batch, params):
    n, d = x.shape
    e = edge_index.shape[1]
    g = 64
    src = edge_index[0]
    dst = edge_index[1]
    h = x
    pooled = []
    for p in params:
        hdim = p["W2"].shape[1]
        agg = _sc_scatter_add(h, src, dst, n=n, e=e, d=h.shape[1])
        h, pool = _tc_layer(agg, batch, p, n=n, d=h.shape[1], hdim=hdim, g=g)
        pooled.append(pool)
    return jnp.concatenate(pooled, axis=-1)
